# baseline (speedup 1.0000x reference)
import functools

import jax
import jax.numpy as jnp
from jax import lax
from jax.experimental import pallas as pl
from jax.experimental.pallas import tpu as pltpu
from jax.experimental.pallas import tpu_sc as plsc

D = 128
NB = 4096
S = 50
SP = 56
B = NB * S
BP = NB * SP
NC, NS = 2, 16
NW = NC * NS
BPW = B // NW
CH = 128
NCHUNK = BPW // CH
NBUF = 5
OUTER = NCHUNK // NBUF

G = 256
BB = NB // G
PB = B // G

_mesh = plsc.VectorSubcoreMesh(core_axis_name="c", subcore_axis_name="s")


@functools.partial(
    pl.kernel,
    mesh=_mesh,
    out_type=jax.ShapeDtypeStruct((BP, D), jnp.float32),
    scratch_types=(
        [pltpu.VMEM((BPW,), jnp.int32),
         pltpu.VMEM((NCHUNK, CH), jnp.int32)]
        + [pltpu.VMEM((CH, D), jnp.float32)] * NBUF
        + [pltpu.SemaphoreType.DMA] * (2 * NBUF)
    ),
)
def _concept_sc(cidx_hbm, ctab_hbm, out_hbm, idx_v, dst_v, *bufs_and_sems):
    rows = bufs_and_sems[:NBUF]
    gsem = bufs_and_sems[NBUF:2 * NBUF]
    ssem = bufs_and_sems[2 * NBUF:]

    wid = lax.axis_index("s") * NC + lax.axis_index("c")
    base = wid * BPW
    bq = wid * (BPW // S)

    pltpu.sync_copy(cidx_hbm.at[pl.ds(base, BPW)], idx_v)

    lanes = lax.iota(jnp.int32, 16)

    def dst_body(i, carry):
        q0, m0 = carry
        for g in range(CH // 16):
            r = m0 + lanes
            w = jnp.where(r >= S, 1, 0)
            dst = (bq + q0 + w) * SP + (r - S * w)
            dst_v[i, pl.ds(16 * g, 16)] = dst
            m1 = m0 + 16
            wrap = jnp.where(m1 >= S, 1, 0)
            q0 = q0 + wrap
            m0 = m1 - S * wrap
        return q0, m0

    lax.fori_loop(0, NCHUNK, dst_body, (jnp.int32(0), jnp.int32(0)))

    for b in range(NBUF):
        pltpu.async_copy(ctab_hbm.at[idx_v.at[pl.ds(b * CH, CH)]],
                         rows[b], gsem[b])

    def outer(k, carry):
        for b in range(NBUF):
            i = k * NBUF + b
            pltpu.make_async_copy(ctab_hbm.at[pl.ds(0, CH)],
                                  rows[b], gsem[b]).wait()
            pltpu.async_copy(rows[b], out_hbm.at[dst_v.at[i]], ssem[b])
            pltpu.make_async_copy(rows[b], out_hbm.at[pl.ds(0, CH)],
                                  ssem[b]).wait()
            nxt = i + NBUF

            @pl.when(nxt < NCHUNK)
            def _fire():
                pltpu.async_copy(
                    ctab_hbm.at[idx_v.at[pl.ds(nxt * CH, CH)]],
                    rows[b], gsem[b])
        return carry

    lax.fori_loop(0, OUTER, outer, 0)


def _tc_body(ridx_ref, thi_ref, cin_ref, cout_ref, rout_ref, scr_ref):
    idx = ridx_ref[0]
    idxb = jnp.broadcast_to(idx, (D, PB))
    kio = lax.broadcasted_iota(jnp.int32, (D, PB), 0)
    ohT = (kio == idxb).astype(jnp.bfloat16)
    scr_ref[...] = lax.dot_general(ohT, thi_ref[...],
                                   (((0,), (0,)), ((), ())),
                                   preferred_element_type=jnp.float32)
    for j in range(BB):
        cout_ref[j] = cin_ref[pl.ds(j * SP, S), :]
        rout_ref[j] = scr_ref[pl.ds(j * S, S), :]


_relay_tc = pl.pallas_call(
    _tc_body,
    grid=(G,),
    in_specs=[
        pl.BlockSpec((1, 1, PB), lambda i: (i, 0, 0)),
        pl.BlockSpec((D, D), lambda i: (0, 0)),
        pl.BlockSpec((BB * SP, D), lambda i: (i, 0)),
    ],
    out_specs=[
        pl.BlockSpec((BB, S, D), lambda i: (i, 0, 0)),
        pl.BlockSpec((BB, S, D), lambda i: (i, 0, 0)),
    ],
    out_shape=[
        jax.ShapeDtypeStruct((NB, S, D), jnp.float32),
        jax.ShapeDtypeStruct((NB, S, D), jnp.float32),
    ],
    scratch_shapes=[pltpu.VMEM((PB, D), jnp.float32)],
)


def kernel(concept_inp, relation_inp, concept_table, relation_table):
    cidx = concept_inp.reshape(-1).astype(jnp.int32)
    ridx3 = relation_inp.reshape(G, 1, PB).astype(jnp.int32)
    tpad = jnp.pad(relation_table, ((0, D - relation_table.shape[0]), (0, 0)))
    thi = tpad.astype(jnp.bfloat16)
    c_pad = _concept_sc(cidx, concept_table)
    cout, rout = _relay_tc(ridx3, thi, c_pad)
    return cout, rout


# speedup vs baseline: 2.1413x; 2.1413x over previous
import functools

import jax
import jax.numpy as jnp
from jax import lax
from jax.experimental import pallas as pl
from jax.experimental.pallas import tpu as pltpu
from jax.experimental.pallas import tpu_sc as plsc

D = 128
NB = 4096
S = 50
B = NB * S
NC, NS = 2, 16
NW = NC * NS
BPW = B // NW
CH = 128
NCHUNK = BPW // CH
NBUF = 5
OUTER = NCHUNK // NBUF

G = 256
BB = NB // G
PB = BB * S

_mesh = plsc.VectorSubcoreMesh(core_axis_name="c", subcore_axis_name="s")


@functools.partial(
    pl.kernel,
    mesh=_mesh,
    out_type=jax.ShapeDtypeStruct((B, D), jnp.float32),
    scratch_types=(
        [pltpu.VMEM((BPW,), jnp.int32),
         pltpu.VMEM((NCHUNK, CH), jnp.int32)]
        + [pltpu.VMEM((CH, D), jnp.float32)] * NBUF
        + [pltpu.SemaphoreType.DMA] * (2 * NBUF)
    ),
)
def _concept_sc(cidx_hbm, ctab_hbm, out_hbm, idx_v, dst_v, *bufs_and_sems):
    rows = bufs_and_sems[:NBUF]
    gsem = bufs_and_sems[NBUF:2 * NBUF]
    ssem = bufs_and_sems[2 * NBUF:]

    wid = lax.axis_index("s") * NC + lax.axis_index("c")
    base = wid * BPW
    bq = wid * (BPW // S)

    pltpu.sync_copy(cidx_hbm.at[pl.ds(base, BPW)], idx_v)

    lanes = lax.iota(jnp.int32, 16)

    def dst_body(i, carry):
        q0, m0 = carry
        for g in range(CH // 16):
            r = m0 + lanes
            w = jnp.where(r >= S, 1, 0)
            dst = (r - S * w) * NB + (bq + q0 + w)
            dst_v[i, pl.ds(16 * g, 16)] = dst
            m1 = m0 + 16
            wrap = jnp.where(m1 >= S, 1, 0)
            q0 = q0 + wrap
            m0 = m1 - S * wrap
        return q0, m0

    lax.fori_loop(0, NCHUNK, dst_body, (jnp.int32(0), jnp.int32(0)))

    for b in range(NBUF):
        pltpu.async_copy(ctab_hbm.at[idx_v.at[pl.ds(b * CH, CH)]],
                         rows[b], gsem[b])

    def outer(k, carry):
        for b in range(NBUF):
            i = k * NBUF + b
            pltpu.make_async_copy(ctab_hbm.at[pl.ds(0, CH)],
                                  rows[b], gsem[b]).wait()
            pltpu.async_copy(rows[b], out_hbm.at[dst_v.at[i]], ssem[b])
            pltpu.make_async_copy(rows[b], out_hbm.at[pl.ds(0, CH)],
                                  ssem[b]).wait()
            nxt = i + NBUF

            @pl.when(nxt < NCHUNK)
            def _fire():
                pltpu.async_copy(
                    ctab_hbm.at[idx_v.at[pl.ds(nxt * CH, CH)]],
                    rows[b], gsem[b])
        return carry

    lax.fori_loop(0, OUTER, outer, 0)


def _tc_body(ridx_ref, thi_ref, rout_ref):
    idx = ridx_ref[0]
    idxb = jnp.broadcast_to(idx, (D, PB))
    kio = lax.broadcasted_iota(jnp.int32, (D, PB), 0)
    ohT = (kio == idxb).astype(jnp.bfloat16)
    rows = lax.dot_general(ohT, thi_ref[...],
                           (((0,), (0,)), ((), ())),
                           preferred_element_type=jnp.float32)
    rout_ref[...] = rows.reshape(S, BB, D)


_relation_tc = pl.pallas_call(
    _tc_body,
    grid=(G,),
    in_specs=[
        pl.BlockSpec((1, 1, PB), lambda i: (i, 0, 0)),
        pl.BlockSpec((D, D), lambda i: (0, 0)),
    ],
    out_specs=pl.BlockSpec((S, BB, D), lambda i: (0, i, 0)),
    out_shape=jax.ShapeDtypeStruct((S, NB, D), jnp.float32),
)


def kernel(concept_inp, relation_inp, concept_table, relation_table):
    cidx = concept_inp.reshape(-1).astype(jnp.int32)
    ridxT3 = (relation_inp.reshape(G, BB, S).transpose(0, 2, 1)
              .reshape(G, 1, PB).astype(jnp.int32))
    tpad = jnp.pad(relation_table, ((0, D - relation_table.shape[0]), (0, 0)))
    thi = tpad.astype(jnp.bfloat16)
    c_sm = _concept_sc(cidx, concept_table)
    routT = _relation_tc(ridxT3, thi)
    cout = c_sm.reshape(S, NB, D).transpose(1, 0, 2)
    rout = routT.transpose(1, 0, 2)
    return cout, rout


# speedup vs baseline: 2.9308x; 1.3687x over previous
import functools

import jax
import jax.numpy as jnp
from jax import lax
from jax.experimental import pallas as pl
from jax.experimental.pallas import tpu as pltpu
from jax.experimental.pallas import tpu_sc as plsc

D = 128
NB = 4096
S = 50
B = NB * S
NC, NS = 2, 16
NW = NC * NS
BPW = B // NW
CH = 128
NCHUNK = BPW // CH
NBUF = 5
OUTER = NCHUNK // NBUF

G = 128
BB = NB // G
PB = BB * S

_mesh = plsc.VectorSubcoreMesh(core_axis_name="c", subcore_axis_name="s")


@functools.partial(
    pl.kernel,
    mesh=_mesh,
    out_type=jax.ShapeDtypeStruct((B, D), jnp.float32),
    scratch_types=(
        [pltpu.VMEM((BPW,), jnp.int32),
         pltpu.VMEM((NCHUNK, CH), jnp.int32)]
        + [pltpu.VMEM((CH, D), jnp.float32)] * NBUF
        + [pltpu.SemaphoreType.DMA] * (2 * NBUF)
    ),
)
def _concept_sc(cidx_hbm, ctab_hbm, out_hbm, idx_v, dst_v, *bufs_and_sems):
    rows = bufs_and_sems[:NBUF]
    gsem = bufs_and_sems[NBUF:2 * NBUF]
    ssem = bufs_and_sems[2 * NBUF:]

    wid = lax.axis_index("s") * NC + lax.axis_index("c")
    base = wid * BPW
    bq = wid * (BPW // S)

    pltpu.sync_copy(cidx_hbm.at[pl.ds(base, BPW)], idx_v)

    lanes = lax.iota(jnp.int32, 16)

    def dst_body(i, carry):
        q0, m0 = carry
        for g in range(CH // 16):
            r = m0 + lanes
            w = jnp.where(r >= S, 1, 0)
            dst = (r - S * w) * NB + (bq + q0 + w)
            dst_v[i, pl.ds(16 * g, 16)] = dst
            m1 = m0 + 16
            wrap = jnp.where(m1 >= S, 1, 0)
            q0 = q0 + wrap
            m0 = m1 - S * wrap
        return q0, m0

    lax.fori_loop(0, NCHUNK, dst_body, (jnp.int32(0), jnp.int32(0)))

    for b in range(NBUF):
        pltpu.async_copy(ctab_hbm.at[idx_v.at[pl.ds(b * CH, CH)]],
                         rows[b], gsem[b])

    def outer(k, carry):
        for b in range(NBUF):
            i = k * NBUF + b
            pltpu.make_async_copy(ctab_hbm.at[pl.ds(0, CH)],
                                  rows[b], gsem[b]).wait()
            pltpu.async_copy(rows[b], out_hbm.at[dst_v.at[i]], ssem[b])
            pltpu.make_async_copy(rows[b], out_hbm.at[pl.ds(0, CH)],
                                  ssem[b]).wait()
            nxt = i + NBUF

            @pl.when(nxt < NCHUNK)
            def _fire():
                pltpu.async_copy(
                    ctab_hbm.at[idx_v.at[pl.ds(nxt * CH, CH)]],
                    rows[b], gsem[b])
        return carry

    lax.fori_loop(0, OUTER, outer, 0)


def _tc_body(ridx_ref, thi_ref, rout_ref):
    idx = ridx_ref[0]
    idxb = jnp.broadcast_to(idx, (D, PB))
    kio = lax.broadcasted_iota(jnp.int32, (D, PB), 0)
    ohT = (kio == idxb).astype(jnp.bfloat16)
    rows = lax.dot_general(ohT, thi_ref[...],
                           (((0,), (0,)), ((), ())),
                           preferred_element_type=jnp.float32)
    rout_ref[...] = rows.reshape(S, BB, D)


_relation_tc = pl.pallas_call(
    _tc_body,
    grid=(G,),
    in_specs=[
        pl.BlockSpec((1, 1, PB), lambda i: (i, 0, 0)),
        pl.BlockSpec((D, D), lambda i: (0, 0)),
    ],
    out_specs=pl.BlockSpec((S, BB, D), lambda i: (0, i, 0)),
    out_shape=jax.ShapeDtypeStruct((S, NB, D), jnp.float32),
)


def kernel(concept_inp, relation_inp, concept_table, relation_table):
    cidx = concept_inp.reshape(-1).astype(jnp.int32)
    ridxT3 = (relation_inp.reshape(G, BB, S).transpose(0, 2, 1)
              .reshape(G, 1, PB).astype(jnp.int32))
    tpad = jnp.pad(relation_table, ((0, D - relation_table.shape[0]), (0, 0)))
    thi = tpad.astype(jnp.bfloat16)
    c_sm = _concept_sc(cidx, concept_table)
    routT = _relation_tc(ridxT3, thi)
    cout = c_sm.reshape(S, NB, D).transpose(1, 0, 2)
    rout = routT.transpose(1, 0, 2)
    return cout, rout


# speedup vs baseline: 3.3967x; 1.1590x over previous
import functools

import jax
import jax.numpy as jnp
from jax import lax
from jax.experimental import pallas as pl
from jax.experimental.pallas import tpu as pltpu
from jax.experimental.pallas import tpu_sc as plsc

D = 128
NB = 4096
S = 50
B = NB * S
NC, NS = 2, 16
NW = NC * NS
BPW = B // NW
CH = 128
NCHUNK = BPW // CH
NBUF = 5
OUTER = NCHUNK // NBUF

G = 64
BB = NB // G
PB = BB * S

_mesh = plsc.VectorSubcoreMesh(core_axis_name="c", subcore_axis_name="s")


@functools.partial(
    pl.kernel,
    mesh=_mesh,
    out_type=jax.ShapeDtypeStruct((B, D), jnp.float32),
    scratch_types=(
        [pltpu.VMEM((BPW,), jnp.int32),
         pltpu.VMEM((NCHUNK, CH), jnp.int32)]
        + [pltpu.VMEM((CH, D), jnp.float32)] * NBUF
        + [pltpu.SemaphoreType.DMA] * (2 * NBUF)
    ),
)
def _concept_sc(cidx_hbm, ctab_hbm, out_hbm, idx_v, dst_v, *bufs_and_sems):
    rows = bufs_and_sems[:NBUF]
    gsem = bufs_and_sems[NBUF:2 * NBUF]
    ssem = bufs_and_sems[2 * NBUF:]

    wid = lax.axis_index("s") * NC + lax.axis_index("c")
    base = wid * BPW
    bq = wid * (BPW // S)

    pltpu.sync_copy(cidx_hbm.at[pl.ds(base, BPW)], idx_v)

    lanes = lax.iota(jnp.int32, 16)

    def dst_body(i, carry):
        q0, m0 = carry
        for g in range(CH // 16):
            r = m0 + lanes
            w = jnp.where(r >= S, 1, 0)
            dst = (r - S * w) * NB + (bq + q0 + w)
            dst_v[i, pl.ds(16 * g, 16)] = dst
            m1 = m0 + 16
            wrap = jnp.where(m1 >= S, 1, 0)
            q0 = q0 + wrap
            m0 = m1 - S * wrap
        return q0, m0

    lax.fori_loop(0, NCHUNK, dst_body, (jnp.int32(0), jnp.int32(0)))

    for b in range(NBUF):
        pltpu.async_copy(ctab_hbm.at[idx_v.at[pl.ds(b * CH, CH)]],
                         rows[b], gsem[b])

    def outer(k, carry):
        for b in range(NBUF):
            i = k * NBUF + b
            pltpu.make_async_copy(ctab_hbm.at[pl.ds(0, CH)],
                                  rows[b], gsem[b]).wait()
            pltpu.async_copy(rows[b], out_hbm.at[dst_v.at[i]], ssem[b])
            pltpu.make_async_copy(rows[b], out_hbm.at[pl.ds(0, CH)],
                                  ssem[b]).wait()
            nxt = i + NBUF

            @pl.when(nxt < NCHUNK)
            def _fire():
                pltpu.async_copy(
                    ctab_hbm.at[idx_v.at[pl.ds(nxt * CH, CH)]],
                    rows[b], gsem[b])
        return carry

    lax.fori_loop(0, OUTER, outer, 0)


def _tc_body(ridx_ref, thi_ref, rout_ref):
    idx = ridx_ref[0]
    idxb = jnp.broadcast_to(idx, (D, PB))
    kio = lax.broadcasted_iota(jnp.int32, (D, PB), 0)
    ohT = (kio == idxb).astype(jnp.bfloat16)
    rows = lax.dot_general(ohT, thi_ref[...],
                           (((0,), (0,)), ((), ())),
                           preferred_element_type=jnp.float32)
    rout_ref[...] = rows.reshape(S, BB, D)


_relation_tc = pl.pallas_call(
    _tc_body,
    grid=(G,),
    in_specs=[
        pl.BlockSpec((1, 1, PB), lambda i: (i, 0, 0)),
        pl.BlockSpec((D, D), lambda i: (0, 0)),
    ],
    out_specs=pl.BlockSpec((S, BB, D), lambda i: (0, i, 0)),
    out_shape=jax.ShapeDtypeStruct((S, NB, D), jnp.float32),
)


def kernel(concept_inp, relation_inp, concept_table, relation_table):
    cidx = concept_inp.reshape(-1).astype(jnp.int32)
    ridxT3 = (relation_inp.reshape(G, BB, S).transpose(0, 2, 1)
              .reshape(G, 1, PB).astype(jnp.int32))
    tpad = jnp.pad(relation_table, ((0, D - relation_table.shape[0]), (0, 0)))
    thi = tpad.astype(jnp.bfloat16)
    c_sm = _concept_sc(cidx, concept_table)
    routT = _relation_tc(ridxT3, thi)
    cout = c_sm.reshape(S, NB, D).transpose(1, 0, 2)
    rout = routT.transpose(1, 0, 2)
    return cout, rout


# speedup vs baseline: 3.4795x; 1.0244x over previous
import functools

import jax
import jax.numpy as jnp
from jax import lax
from jax.experimental import pallas as pl
from jax.experimental.pallas import tpu as pltpu
from jax.experimental.pallas import tpu_sc as plsc

D = 128
NB = 4096
S = 50
B = NB * S
NC, NS = 2, 16
NW = NC * NS
BPW = B // NW
CH = 128
NCHUNK = BPW // CH
NBUF = 5
OUTER = NCHUNK // NBUF

G = 32
BB = NB // G
PB = BB * S

_mesh = plsc.VectorSubcoreMesh(core_axis_name="c", subcore_axis_name="s")


@functools.partial(
    pl.kernel,
    mesh=_mesh,
    out_type=jax.ShapeDtypeStruct((B, D), jnp.float32),
    scratch_types=(
        [pltpu.VMEM((BPW,), jnp.int32),
         pltpu.VMEM((NCHUNK, CH), jnp.int32)]
        + [pltpu.VMEM((CH, D), jnp.float32)] * NBUF
        + [pltpu.SemaphoreType.DMA] * (2 * NBUF)
    ),
)
def _concept_sc(cidx_hbm, ctab_hbm, out_hbm, idx_v, dst_v, *bufs_and_sems):
    rows = bufs_and_sems[:NBUF]
    gsem = bufs_and_sems[NBUF:2 * NBUF]
    ssem = bufs_and_sems[2 * NBUF:]

    wid = lax.axis_index("s") * NC + lax.axis_index("c")
    base = wid * BPW
    bq = wid * (BPW // S)

    pltpu.sync_copy(cidx_hbm.at[pl.ds(base, BPW)], idx_v)

    lanes = lax.iota(jnp.int32, 16)

    def dst_body(i, carry):
        q0, m0 = carry
        for g in range(CH // 16):
            r = m0 + lanes
            w = jnp.where(r >= S, 1, 0)
            dst = (r - S * w) * NB + (bq + q0 + w)
            dst_v[i, pl.ds(16 * g, 16)] = dst
            m1 = m0 + 16
            wrap = jnp.where(m1 >= S, 1, 0)
            q0 = q0 + wrap
            m0 = m1 - S * wrap
        return q0, m0

    lax.fori_loop(0, NCHUNK, dst_body, (jnp.int32(0), jnp.int32(0)))

    for b in range(NBUF):
        pltpu.async_copy(ctab_hbm.at[idx_v.at[pl.ds(b * CH, CH)]],
                         rows[b], gsem[b])

    def outer(k, carry):
        for b in range(NBUF):
            i = k * NBUF + b
            pltpu.make_async_copy(ctab_hbm.at[pl.ds(0, CH)],
                                  rows[b], gsem[b]).wait()
            pltpu.async_copy(rows[b], out_hbm.at[dst_v.at[i]], ssem[b])
            pltpu.make_async_copy(rows[b], out_hbm.at[pl.ds(0, CH)],
                                  ssem[b]).wait()
            nxt = i + NBUF

            @pl.when(nxt < NCHUNK)
            def _fire():
                pltpu.async_copy(
                    ctab_hbm.at[idx_v.at[pl.ds(nxt * CH, CH)]],
                    rows[b], gsem[b])
        return carry

    lax.fori_loop(0, OUTER, outer, 0)


def _tc_body(ridx_ref, thi_ref, rout_ref):
    idx = ridx_ref[0]
    idxb = jnp.broadcast_to(idx, (D, PB))
    kio = lax.broadcasted_iota(jnp.int32, (D, PB), 0)
    ohT = (kio == idxb).astype(jnp.bfloat16)
    rows = lax.dot_general(ohT, thi_ref[...],
                           (((0,), (0,)), ((), ())),
                           preferred_element_type=jnp.float32)
    rout_ref[...] = rows.reshape(S, BB, D)


_relation_tc = pl.pallas_call(
    _tc_body,
    grid=(G,),
    in_specs=[
        pl.BlockSpec((1, 1, PB), lambda i: (i, 0, 0)),
        pl.BlockSpec((D, D), lambda i: (0, 0)),
    ],
    out_specs=pl.BlockSpec((S, BB, D), lambda i: (0, i, 0)),
    out_shape=jax.ShapeDtypeStruct((S, NB, D), jnp.float32),
)


def kernel(concept_inp, relation_inp, concept_table, relation_table):
    cidx = concept_inp.reshape(-1).astype(jnp.int32)
    ridxT3 = (relation_inp.reshape(G, BB, S).transpose(0, 2, 1)
              .reshape(G, 1, PB).astype(jnp.int32))
    tpad = jnp.pad(relation_table, ((0, D - relation_table.shape[0]), (0, 0)))
    thi = tpad.astype(jnp.bfloat16)
    c_sm = _concept_sc(cidx, concept_table)
    routT = _relation_tc(ridxT3, thi)
    cout = c_sm.reshape(S, NB, D).transpose(1, 0, 2)
    rout = routT.transpose(1, 0, 2)
    return cout, rout
